# baseline (speedup 1.0000x reference)
import functools

import jax
import jax.numpy as jnp
from jax import lax
from jax.experimental import pallas as pl
from jax.experimental.pallas import tpu as pltpu
from jax.experimental.pallas import tpu_sc as plsc

_NC = 2
_NS = 16
_NW = _NC * _NS

_EMBED = 64
_CH = 128
_NB = 4


@functools.lru_cache(maxsize=None)
def _make_gather(vocab, batch, seq):
    units = (batch // _CH) * seq
    assert units % _NW == 0
    u_per_w = units // _NW
    assert u_per_w % _NB == 0
    nbc = batch // _CH

    mesh = plsc.VectorSubcoreMesh(core_axis_name="c", subcore_axis_name="s")

    scratch = [pltpu.VMEM((u_per_w, _CH), jnp.int32)]
    scratch += [pltpu.VMEM((_CH, _EMBED), jnp.float32) for _ in range(_NB)]
    scratch += [pltpu.VMEM((_EMBED // 8, 8, _CH + 1), jnp.float32)
                for _ in range(_NB)]
    scratch += [pltpu.SemaphoreType.DMA for _ in range(2 * _NB)]

    @functools.partial(
        pl.kernel,
        mesh=mesh,
        out_type=jax.ShapeDtypeStruct(
            (seq, _EMBED // 8, nbc, 8, _CH), jnp.float32),
        scratch_types=scratch,
        compiler_params=pltpu.CompilerParams(use_tc_tiling_on_sc=False,
                                             needs_layout_passes=False),
    )
    def k(table_hbm, idx_hbm, out_hbm, idx_v, *bufs_and_sems):
        rows = bufs_and_sems[:_NB]
        trs = bufs_and_sems[_NB:2 * _NB]
        gsem = bufs_and_sems[2 * _NB:3 * _NB]
        wsem = bufs_and_sems[3 * _NB:]
        wid = lax.axis_index("s") * _NC + lax.axis_index("c")
        u0 = wid * u_per_w

        pltpu.sync_copy(idx_hbm.at[pl.ds(u0, u_per_w)], idx_v)

        def fire_gather(g, b):
            pltpu.async_copy(table_hbm.at[idx_v.at[g]], rows[b], gsem[b])

        def wait_gather(b):
            pltpu.make_async_copy(
                table_hbm.at[idx_v.at[0]], rows[b], gsem[b]).wait()

        def tr_src(b):
            return trs[b].at[:, :, pl.ds(0, _CH)]

        def fire_write(j, bc, b):
            pltpu.async_copy(tr_src(b), out_hbm.at[j, :, bc, :, :], wsem[b])

        def wait_write(b):
            pltpu.make_async_copy(tr_src(b), out_hbm.at[0, :, 0, :, :],
                                  wsem[b]).wait()

        iota16 = lax.iota(jnp.int32, 16)
        zero16 = iota16 * 0
        e_ids = [e0 + iota16 for e0 in range(0, _EMBED, 16)]

        def transpose(b):
            rows_b, tr_b = rows[b], trs[b]

            def body_bi(i, carry):
                for sub in range(8):
                    bi = i * 8 + sub
                    bis = jnp.full((16,), bi, jnp.int32)
                    for g in range(_EMBED // 16):
                        v = rows_b[bi, pl.ds(g * 16, 16)]
                        plsc.store_scatter(
                            tr_b, [zero16, e_ids[g], bis], v)
                return carry

            lax.fori_loop(0, _CH // 8, body_bi, 0)

        for b in range(_NB):
            fire_gather(b, b)

        def outer(i, carry):
            for b in range(_NB):
                g = i * _NB + b
                wait_gather(b)

                @pl.when(g >= _NB)
                def _():
                    wait_write(b)

                transpose(b)
                u = u0 + g
                fire_write(u // nbc, u % nbc, b)

                @pl.when(g + _NB < u_per_w)
                def _():
                    fire_gather(g + _NB, b)
            return carry

        lax.fori_loop(0, u_per_w // _NB, outer, 0)

        for b in range(_NB):
            wait_write(b)

    return k


def kernel(questions_tensor, table):
    batch, seq = questions_tensor.shape
    vocab, embed = table.shape
    idx = questions_tensor.T.reshape((batch // _CH) * seq, _CH)
    out5 = _make_gather(vocab, batch, seq)(table, idx)
    return jnp.transpose(out5, (2, 4, 0, 1, 3)).reshape(batch, seq, embed)


# speedup vs baseline: 1.1326x; 1.1326x over previous
import functools

import jax
import jax.numpy as jnp
from jax import lax
from jax.experimental import pallas as pl
from jax.experimental.pallas import tpu as pltpu
from jax.experimental.pallas import tpu_sc as plsc

_NC = 2
_NS = 16
_NW = _NC * _NS

_EMBED = 64
_CH = 128
_NB = 4


@functools.lru_cache(maxsize=None)
def _make_gather(vocab, batch, seq):
    units = (batch // _CH) * seq
    assert units % _NW == 0
    u_per_w = units // _NW
    assert u_per_w % _NB == 0
    nbc = batch // _CH

    mesh = plsc.VectorSubcoreMesh(core_axis_name="c", subcore_axis_name="s")

    scratch = [pltpu.VMEM((u_per_w, _CH), jnp.int32)]
    scratch += [pltpu.VMEM((_CH, _EMBED), jnp.float32) for _ in range(_NB)]
    scratch += [pltpu.VMEM((_EMBED // 8, 8, _CH + 1), jnp.float32)
                for _ in range(_NB)]
    scratch += [pltpu.SemaphoreType.DMA for _ in range(2 * _NB)]

    @functools.partial(
        pl.kernel,
        mesh=mesh,
        out_type=jax.ShapeDtypeStruct(
            (seq, _EMBED // 8, nbc, 8, _CH), jnp.float32),
        scratch_types=scratch,
        compiler_params=pltpu.CompilerParams(use_tc_tiling_on_sc=False,
                                             needs_layout_passes=False),
    )
    def k(table_hbm, idx_hbm, out_hbm, idx_v, *bufs_and_sems):
        rows = bufs_and_sems[:_NB]
        trs = bufs_and_sems[_NB:2 * _NB]
        gsem = bufs_and_sems[2 * _NB:3 * _NB]
        wsem = bufs_and_sems[3 * _NB:]
        wid = lax.axis_index("s") * _NC + lax.axis_index("c")
        u0 = wid * u_per_w

        pltpu.sync_copy(idx_hbm.at[pl.ds(u0, u_per_w)], idx_v)

        def fire_gather(g, b):
            pltpu.async_copy(table_hbm.at[idx_v.at[g]], rows[b], gsem[b])

        def wait_gather(b):
            pltpu.make_async_copy(
                table_hbm.at[idx_v.at[0]], rows[b], gsem[b]).wait()

        def tr_src(b):
            return trs[b].at[:, :, pl.ds(0, _CH)]

        def fire_write(j, bc, b):
            pltpu.async_copy(tr_src(b), out_hbm.at[j, :, bc, :, :], wsem[b])

        def wait_write(b):
            pltpu.make_async_copy(tr_src(b), out_hbm.at[0, :, 0, :, :],
                                  wsem[b]).wait()

        iota16 = lax.iota(jnp.int32, 16)
        zero16 = iota16 * 0
        e_ids = [e0 + iota16 for e0 in range(0, _EMBED, 16)]

        def transpose(b):
            rows_b, tr_b = rows[b], trs[b]

            def body_bi(i, carry):
                for sub in range(8):
                    bi = i * 8 + sub
                    bis = jnp.full((16,), bi, jnp.int32)
                    for g in range(_EMBED // 16):
                        v = rows_b[bi, pl.ds(g * 16, 16)]
                        plsc.store_scatter(
                            tr_b, [zero16, e_ids[g], bis], v)
                return carry

            lax.fori_loop(0, _CH // 8, body_bi, 0)

        for b in range(_NB):
            fire_gather(b, b)

        def outer(i, carry):
            for b in range(_NB):
                g = i * _NB + b
                wait_gather(b)

                @pl.when(g >= _NB)
                def _():
                    wait_write(b)

                transpose(b)
                u = u0 + g
                fire_write(u // nbc, u % nbc, b)

                @pl.when(g + _NB < u_per_w)
                def _():
                    fire_gather(g + _NB, b)
            return carry

        lax.fori_loop(0, u_per_w // _NB, outer, 0)

        for b in range(_NB):
            wait_write(b)

    return k


_KG = 8
_ENB = 4


@functools.lru_cache(maxsize=None)
def _make_extract(vocab):
    groups = vocab // 8
    total = groups // _KG
    slots = ((total + _NW - 1) // _NW + _ENB - 1) // _ENB * _ENB

    mesh = plsc.VectorSubcoreMesh(core_axis_name="c", subcore_axis_name="s")

    scratch = [pltpu.VMEM((_KG, 8, _EMBED), jnp.float32)
               for _ in range(_ENB)]
    scratch += [pltpu.VMEM((_KG * 8 // 2, 128), jnp.float32)
                for _ in range(_ENB)]
    scratch += [pltpu.SemaphoreType.DMA for _ in range(2 * _ENB)]

    @functools.partial(
        pl.kernel,
        mesh=mesh,
        out_type=jax.ShapeDtypeStruct((vocab // 2, 128), jnp.float32),
        scratch_types=scratch,
        compiler_params=pltpu.CompilerParams(use_tc_tiling_on_sc=True,
                                             needs_layout_passes=False),
    )
    def ka(t3_hbm, out_hbm, *bufs_and_sems):
        pads = bufs_and_sems[:_ENB]
        stages = bufs_and_sems[_ENB:2 * _ENB]
        gsem = bufs_and_sems[2 * _ENB:3 * _ENB]
        wsem = bufs_and_sems[3 * _ENB:]
        wid = lax.axis_index("s") * _NC + lax.axis_index("c")

        def fire_in(s, b):
            c = s * _NW + wid

            @pl.when(c < total)
            def _():
                pltpu.async_copy(t3_hbm.at[pl.ds(c * _KG, _KG)],
                                 pads[b], gsem[b])

        def wait_in(b):
            pltpu.make_async_copy(t3_hbm.at[pl.ds(0, _KG)],
                                  pads[b], gsem[b]).wait()

        def wait_out(b):
            pltpu.make_async_copy(stages[b],
                                  out_hbm.at[pl.ds(0, _KG * 4)],
                                  wsem[b]).wait()

        def extract(b):
            pad_b, stage_b = pads[b], stages[b]
            for k in range(_KG):
                for r in range(8):
                    q, p = (k * 8 + r) // 2, (k * 8 + r) % 2
                    for g in range(_EMBED // 16):
                        stage_b[q, pl.ds(64 * p + g * 16, 16)] = (
                            pad_b[k, r, pl.ds(g * 16, 16)])

        for b in range(_ENB):
            fire_in(b, b)

        def outer(i, carry):
            for b in range(_ENB):
                s = i * _ENB + b
                c = s * _NW + wid

                @pl.when(c < total)
                def _():
                    wait_in(b)

                    @pl.when(s >= _ENB)
                    def _():
                        wait_out(b)

                    extract(b)
                    pltpu.async_copy(stages[b],
                                     out_hbm.at[pl.ds(c * _KG * 4, _KG * 4)],
                                     wsem[b])

                fire_in(s + _ENB, b)
            return carry

        lax.fori_loop(0, slots // _ENB, outer, 0)

        for b in range(_ENB):
            wait_out(b)

    return ka


def kernel(questions_tensor, table):
    batch, seq = questions_tensor.shape
    vocab, embed = table.shape
    packed = _make_extract(vocab)(table.reshape(vocab // 8, 8, embed))
    table_lin = packed.reshape(vocab, embed)
    idx = questions_tensor.T.reshape((batch // _CH) * seq, _CH)
    out5 = _make_gather(vocab, batch, seq)(table_lin, idx)
    return jnp.transpose(out5, (2, 4, 0, 1, 3)).reshape(batch, seq, embed)
